# baseline (speedup 1.0000x reference)
import functools

import jax
import jax.numpy as jnp
from jax import lax
from jax.experimental import pallas as pl
from jax.experimental.pallas import tpu as pltpu
from jax.experimental.pallas import tpu_sc as plsc

D = 64
MAXLEN = 200
LANES = 16
NC = 2
NS = 16
NW = NC * NS

GATHER_SPLITS = ((0, 104), (104, 96))
CHUNK_BATCHES = 4


@functools.partial(jax.jit, static_argnums=(3,))
def _sc_embed(seq, token_table, pe_table, batch):
    batches_per_worker = batch // NW
    chunks_per_worker = batches_per_worker // CHUNK_BATCHES

    mesh = plsc.VectorSubcoreMesh(core_axis_name="c", subcore_axis_name="s")

    @functools.partial(
        pl.kernel,
        mesh=mesh,
        out_type=jax.ShapeDtypeStruct((batch, MAXLEN, D), jnp.float32),
        scratch_types=[
            pltpu.VMEM((MAXLEN, D), jnp.float32),
            pltpu.VMEM((CHUNK_BATCHES, MAXLEN), jnp.int32),
            pltpu.VMEM((CHUNK_BATCHES, MAXLEN, D), jnp.float32),
            pltpu.SemaphoreType.DMA,
        ],
        compiler_params=pltpu.CompilerParams(use_tc_tiling_on_sc=False),
    )
    def body(seq_hbm, table_hbm, pe_hbm, out_hbm, pe_v, idx_v, rows_v, sem):
        wid = lax.axis_index("s") * NC + lax.axis_index("c")
        pltpu.sync_copy(pe_hbm, pe_v)
        base_b = wid * batches_per_worker

        def chunk_body(ci, carry):
            b0 = base_b + ci * CHUNK_BATCHES
            pltpu.sync_copy(seq_hbm.at[pl.ds(b0, CHUNK_BATCHES)], idx_v)
            copies = []
            for i in range(CHUNK_BATCHES):
                for (g0, glen) in GATHER_SPLITS:
                    copies.append(pltpu.async_copy(
                        table_hbm.at[idx_v.at[i, pl.ds(g0, glen)]],
                        rows_v.at[i, pl.ds(g0, glen)],
                        sem))
            for c in copies:
                c.wait()

            def add_body(t, c2):
                for j in range(D // LANES):
                    p = pe_v[t, pl.ds(j * LANES, LANES)]
                    for i in range(CHUNK_BATCHES):
                        plsc.addupdate(
                            rows_v.at[i, t, pl.ds(j * LANES, LANES)], p)
                return c2

            lax.fori_loop(0, MAXLEN, add_body, 0, unroll=False)
            pltpu.sync_copy(rows_v, out_hbm.at[pl.ds(b0, CHUNK_BATCHES)])
            return carry

        lax.fori_loop(0, chunks_per_worker, chunk_body, 0, unroll=False)

    return body(seq, token_table, pe_table)


def kernel(seq, token_table, pe_table):
    batch = seq.shape[0]
    return _sc_embed(seq.astype(jnp.int32), token_table, pe_table, batch)


# speedup vs baseline: 1.0194x; 1.0194x over previous
import functools

import jax
import jax.numpy as jnp
from jax import lax
from jax.experimental import pallas as pl
from jax.experimental.pallas import tpu as pltpu
from jax.experimental.pallas import tpu_sc as plsc

D = 64
DPAD = 128
MAXLEN = 200
LANES = 16
NC = 2
NS = 16
NW = NC * NS

GATHER_SPLITS = ((0, 104), (104, 96))
CHUNK_BATCHES = 4
CHUNK_ROWS = CHUNK_BATCHES * MAXLEN


@functools.partial(jax.jit, static_argnums=(3,))
def _sc_embed(seq, table128, pe_table, batch):
    batches_per_worker = batch // NW
    chunks_per_worker = batches_per_worker // CHUNK_BATCHES

    mesh = plsc.VectorSubcoreMesh(core_axis_name="c", subcore_axis_name="s")

    @functools.partial(
        pl.kernel,
        mesh=mesh,
        out_type=jax.ShapeDtypeStruct((batch * MAXLEN, D), jnp.float32),
        scratch_types=[
            pltpu.VMEM((MAXLEN, D), jnp.float32),
            pltpu.VMEM((CHUNK_BATCHES, MAXLEN), jnp.int32),
            pltpu.VMEM((CHUNK_ROWS, DPAD), jnp.float32),
            pltpu.SemaphoreType.DMA,
        ],
        compiler_params=pltpu.CompilerParams(use_tc_tiling_on_sc=False),
    )
    def body(seq_hbm, table_hbm, pe_hbm, out_hbm, pe_v, idx_v, rows_v, sem):
        wid = lax.axis_index("s") * NC + lax.axis_index("c")
        pltpu.sync_copy(pe_hbm, pe_v)
        base_b = wid * batches_per_worker

        def chunk_body(ci, carry):
            b0 = base_b + ci * CHUNK_BATCHES
            row0 = b0 * MAXLEN
            pltpu.sync_copy(seq_hbm.at[pl.ds(b0, CHUNK_BATCHES)], idx_v)
            copies = []
            for i in range(CHUNK_BATCHES):
                for (g0, glen) in GATHER_SPLITS:
                    copies.append(pltpu.async_copy(
                        table_hbm.at[idx_v.at[i, pl.ds(g0, glen)]],
                        rows_v.at[pl.ds(i * MAXLEN + g0, glen)],
                        sem))
            for c in copies:
                c.wait()

            def add_body(t, c2):
                for j in range(D // LANES):
                    p = pe_v[t, pl.ds(j * LANES, LANES)]
                    for i in range(CHUNK_BATCHES):
                        plsc.addupdate(
                            rows_v.at[i * MAXLEN + t, pl.ds(j * LANES, LANES)],
                            p)
                return c2

            lax.fori_loop(0, MAXLEN, add_body, 0, unroll=False)
            pltpu.sync_copy(rows_v.at[:, pl.ds(0, D)],
                            out_hbm.at[pl.ds(row0, CHUNK_ROWS)])
            return carry

        lax.fori_loop(0, chunks_per_worker, chunk_body, 0, unroll=False)

    return body(seq, table128, pe_table)


def kernel(seq, token_table, pe_table):
    batch, maxlen = seq.shape
    table128 = jnp.pad(token_table, ((0, 0), (0, DPAD - D)))
    out = _sc_embed(seq.astype(jnp.int32), table128, pe_table, batch)
    return out.reshape(batch, maxlen, D)


# speedup vs baseline: 1.0355x; 1.0158x over previous
import functools

import jax
import jax.numpy as jnp
from jax import lax
from jax.experimental import pallas as pl
from jax.experimental.pallas import tpu as pltpu
from jax.experimental.pallas import tpu_sc as plsc

D = 64
DPAD = 128
MAXLEN = 200
LANES = 16
NC = 2
NS = 16
NW = NC * NS

GATHER_SPLITS = ((0, 104), (104, 96))
CHUNK_BATCHES = 2
CHUNK_ROWS = CHUNK_BATCHES * MAXLEN
NBUF = 2


@functools.partial(jax.jit, static_argnums=(3,))
def _sc_embed(seq, table128, pe_table, batch):
    batches_per_worker = batch // NW
    chunks_per_worker = batches_per_worker // CHUNK_BATCHES

    mesh = plsc.VectorSubcoreMesh(core_axis_name="c", subcore_axis_name="s")

    @functools.partial(
        pl.kernel,
        mesh=mesh,
        out_type=jax.ShapeDtypeStruct((batch * MAXLEN, D), jnp.float32),
        scratch_types=[
            pltpu.VMEM((MAXLEN, D), jnp.float32),
            pltpu.VMEM((NBUF, CHUNK_BATCHES, MAXLEN), jnp.int32),
            pltpu.VMEM((NBUF, CHUNK_ROWS, DPAD), jnp.float32),
            pltpu.SemaphoreType.DMA,
            pltpu.SemaphoreType.DMA,
            pltpu.SemaphoreType.DMA,
            pltpu.SemaphoreType.DMA,
        ],
        compiler_params=pltpu.CompilerParams(use_tc_tiling_on_sc=False),
    )
    def body(seq_hbm, table_hbm, pe_hbm, out_hbm, pe_v, idx_v, rows_v,
             gsem0, gsem1, osem0, osem1):
        gsem = (gsem0, gsem1)
        osem = (osem0, osem1)
        wid = lax.axis_index("s") * NC + lax.axis_index("c")
        pltpu.sync_copy(pe_hbm, pe_v)
        base_b = wid * batches_per_worker

        def fire(ci, buf):
            b0 = base_b + ci * CHUNK_BATCHES
            pltpu.sync_copy(seq_hbm.at[pl.ds(b0, CHUNK_BATCHES)],
                            idx_v.at[buf])
            handles = []
            for i in range(CHUNK_BATCHES):
                for (g0, glen) in GATHER_SPLITS:
                    handles.append(pltpu.async_copy(
                        table_hbm.at[idx_v.at[buf, i, pl.ds(g0, glen)]],
                        rows_v.at[buf, pl.ds(i * MAXLEN + g0, glen)],
                        gsem[buf]))
            return handles

        out_handles = [None] * NBUF
        gather_handles = fire(0, 0)
        for ci in range(chunks_per_worker):
            buf = ci % NBUF
            nxt = (ci + 1) % NBUF
            if ci + 1 < chunks_per_worker:
                if out_handles[nxt] is not None:
                    out_handles[nxt].wait()
                    out_handles[nxt] = None
                next_handles = fire(ci + 1, nxt)
            else:
                next_handles = None
            for h in gather_handles:
                h.wait()
            gather_handles = next_handles

            def add_body(t, c2, _buf=buf):
                for j in range(D // LANES):
                    p = pe_v[t, pl.ds(j * LANES, LANES)]
                    for i in range(CHUNK_BATCHES):
                        plsc.addupdate(
                            rows_v.at[_buf, i * MAXLEN + t,
                                      pl.ds(j * LANES, LANES)],
                            p)
                return c2

            lax.fori_loop(0, MAXLEN, add_body, 0, unroll=False)
            row0 = (base_b + ci * CHUNK_BATCHES) * MAXLEN
            out_handles[buf] = pltpu.async_copy(
                rows_v.at[buf, :, pl.ds(0, D)],
                out_hbm.at[pl.ds(row0, CHUNK_ROWS)],
                osem[buf])
        for h in out_handles:
            if h is not None:
                h.wait()

    return body(seq, table128, pe_table)


def kernel(seq, token_table, pe_table):
    batch, maxlen = seq.shape
    table128 = jnp.pad(token_table, ((0, 0), (0, DPAD - D)))
    out = _sc_embed(seq.astype(jnp.int32), table128, pe_table, batch)
    return out.reshape(batch, maxlen, D)


# speedup vs baseline: 1.0384x; 1.0028x over previous
import functools

import jax
import jax.numpy as jnp
from jax import lax
from jax.experimental import pallas as pl
from jax.experimental.pallas import tpu as pltpu
from jax.experimental.pallas import tpu_sc as plsc

D = 64
DPAD = 128
MAXLEN = 200
LANES = 16
NC = 2
NS = 16
NW = NC * NS

GATHER_SPLITS = ((0, 104), (104, 96))
CHUNK_BATCHES = 2
CHUNK_ROWS = CHUNK_BATCHES * MAXLEN
NBUF = 2


@functools.partial(jax.jit, static_argnums=(3,))
def _sc_embed(seq, table128, pe_table, batch):
    batches_per_worker = batch // NW
    chunks_per_worker = batches_per_worker // CHUNK_BATCHES

    mesh = plsc.VectorSubcoreMesh(core_axis_name="c", subcore_axis_name="s")

    @functools.partial(
        pl.kernel,
        mesh=mesh,
        out_type=jax.ShapeDtypeStruct((batch * MAXLEN, D), jnp.float32),
        scratch_types=[
            pltpu.VMEM((MAXLEN, D), jnp.float32),
            pltpu.VMEM((NBUF, CHUNK_BATCHES, MAXLEN), jnp.int32),
            pltpu.VMEM((NBUF, CHUNK_ROWS, DPAD), jnp.float32),
            pltpu.SemaphoreType.DMA,
            pltpu.SemaphoreType.DMA,
            pltpu.SemaphoreType.DMA,
            pltpu.SemaphoreType.DMA,
        ],
        compiler_params=pltpu.CompilerParams(use_tc_tiling_on_sc=False),
    )
    def body(seq_hbm, table_hbm, pe_hbm, out_hbm, pe_v, idx_v, rows_v,
             gsem0, gsem1, osem0, osem1):
        gsem = (gsem0, gsem1)
        osem = (osem0, osem1)
        wid = lax.axis_index("s") * NC + lax.axis_index("c")
        pltpu.sync_copy(pe_hbm, pe_v)
        base_b = wid * batches_per_worker

        def fire(ci, buf):
            b0 = base_b + ci * CHUNK_BATCHES
            pltpu.sync_copy(seq_hbm.at[pl.ds(b0, CHUNK_BATCHES)],
                            idx_v.at[buf])
            handles = []
            for i in range(CHUNK_BATCHES):
                for (g0, glen) in GATHER_SPLITS:
                    handles.append(pltpu.async_copy(
                        table_hbm.at[idx_v.at[buf, i, pl.ds(g0, glen)]],
                        rows_v.at[buf, pl.ds(i * MAXLEN + g0, glen)],
                        gsem[buf]))
            return handles

        out_handles = [None] * NBUF
        gather_handles = fire(0, 0)
        for ci in range(chunks_per_worker):
            buf = ci % NBUF
            nxt = (ci + 1) % NBUF
            if ci + 1 < chunks_per_worker:
                if out_handles[nxt] is not None:
                    out_handles[nxt].wait()
                    out_handles[nxt] = None
                next_handles = fire(ci + 1, nxt)
            else:
                next_handles = None
            for h in gather_handles:
                h.wait()
            gather_handles = next_handles
            row0 = (base_b + ci * CHUNK_BATCHES) * MAXLEN
            out_handles[buf] = pltpu.async_copy(
                rows_v.at[buf, :, pl.ds(0, D)],
                out_hbm.at[pl.ds(row0, CHUNK_ROWS)],
                osem[buf])
        for h in out_handles:
            if h is not None:
                h.wait()

    return body(seq, table128, pe_table)


def kernel(seq, token_table, pe_table):
    batch, maxlen = seq.shape
    table128 = jnp.pad(token_table, ((0, 0), (0, DPAD - D)))
    out = _sc_embed(seq.astype(jnp.int32), table128, pe_table, batch)
    return out.reshape(batch, maxlen, D) + pe_table[None]


# speedup vs baseline: 1.0399x; 1.0014x over previous
import functools

import jax
import jax.numpy as jnp
from jax import lax
from jax.experimental import pallas as pl
from jax.experimental.pallas import tpu as pltpu
from jax.experimental.pallas import tpu_sc as plsc

D = 64
DPAD = 128
MAXLEN = 200
LANES = 16
NC = 2
NS = 16
NW = NC * NS

GATHER_SPLITS = ((0, 104), (104, 96))
CHUNK_BATCHES = 2
CHUNK_ROWS = CHUNK_BATCHES * MAXLEN
NBUF = 2


@functools.partial(jax.jit, static_argnums=(3,))
def _sc_embed(seq, table128, pe_table, batch):
    batches_per_worker = batch // NW
    chunks_per_worker = batches_per_worker // CHUNK_BATCHES

    mesh = plsc.VectorSubcoreMesh(core_axis_name="c", subcore_axis_name="s")

    @functools.partial(
        pl.kernel,
        mesh=mesh,
        out_type=jax.ShapeDtypeStruct((batch * MAXLEN, D), jnp.float32),
        scratch_types=[
            pltpu.VMEM((MAXLEN, D), jnp.float32),
            pltpu.VMEM((NBUF, CHUNK_BATCHES, MAXLEN), jnp.int32),
            pltpu.VMEM((NBUF, CHUNK_ROWS, DPAD), jnp.float32),
            pltpu.SemaphoreType.DMA,
            pltpu.SemaphoreType.DMA,
            pltpu.SemaphoreType.DMA,
            pltpu.SemaphoreType.DMA,
        ],
        compiler_params=pltpu.CompilerParams(use_tc_tiling_on_sc=False),
    )
    def body(seq_hbm, table_hbm, pe_hbm, out_hbm, pe_v, idx_v, rows_v,
             gsem0, gsem1, osem0, osem1):
        gsem = (gsem0, gsem1)
        osem = (osem0, osem1)
        wid = lax.axis_index("s") * NC + lax.axis_index("c")
        pltpu.sync_copy(pe_hbm, pe_v)
        base_b = wid * batches_per_worker

        def fire(ci, buf):
            b0 = base_b + ci * CHUNK_BATCHES
            pltpu.sync_copy(seq_hbm.at[pl.ds(b0, CHUNK_BATCHES)],
                            idx_v.at[buf])
            handles = []
            for i in range(CHUNK_BATCHES):
                for (g0, glen) in GATHER_SPLITS:
                    handles.append(pltpu.async_copy(
                        table_hbm.at[idx_v.at[buf, i, pl.ds(g0, glen)]],
                        rows_v.at[buf, pl.ds(i * MAXLEN + g0, glen)],
                        gsem[buf]))
            return handles

        out_handles = [None] * NBUF
        gather_handles = fire(0, 0)
        for ci in range(chunks_per_worker):
            buf = ci % NBUF
            nxt = (ci + 1) % NBUF
            if ci + 1 < chunks_per_worker:
                if out_handles[nxt] is not None:
                    out_handles[nxt].wait()
                    out_handles[nxt] = None
                next_handles = fire(ci + 1, nxt)
            else:
                next_handles = None
            for h in gather_handles:
                h.wait()
            gather_handles = next_handles
            row0 = (base_b + ci * CHUNK_BATCHES) * MAXLEN
            out_handles[buf] = pltpu.async_copy(
                rows_v.at[buf, :, pl.ds(0, D)],
                out_hbm.at[pl.ds(row0, CHUNK_ROWS)],
                osem[buf])
        for h in out_handles:
            if h is not None:
                h.wait()

    return body(seq, table128, pe_table)


def kernel(seq, token_table, pe_table):
    batch, maxlen = seq.shape
    table128 = jnp.pad(token_table, ((0, 0), (0, DPAD - D)))
    out = _sc_embed(seq.astype(jnp.int32), table128, pe_table, batch)
    out_t = out.reshape(batch, maxlen, D).transpose(1, 2, 0)
    out_t = out_t + pe_table[:, :, None]
    return out_t.transpose(2, 0, 1)
